# baseline (speedup 1.0000x reference)
import functools

import jax
import jax.numpy as jnp
from jax.experimental import pallas as pl
from jax.experimental.pallas import tpu as pltpu

_BN = 8


def _stats_kernel(x_ref, g_ref, t2_ref, tc0_ref, tc1_ref, stats_ref, *,
                  BN, D, P, L):
    x = x_ref[...]
    g = g_ref[...]
    zero = jnp.float32(0.0)

    g2 = g.reshape(BN * D * P, L).astype(jnp.bfloat16)
    dims = (((1,), (0,)), ((), ()))
    a = jax.lax.dot_general(g2, t2_ref[...], dims,
                            preferred_element_type=jnp.float32)
    b0 = jax.lax.dot_general(g2, tc0_ref[...], dims,
                             preferred_element_type=jnp.float32)
    b1 = jax.lax.dot_general(g2, tc1_ref[...], dims,
                             preferred_element_type=jnp.float32)
    a = a.reshape(BN, D, P, L)
    b0 = b0.reshape(BN, D, P, L)
    b1 = b1.reshape(BN, D, P, L)

    dd = jax.lax.broadcasted_iota(jnp.int32, (BN, D, P, L), 1)
    pp = jax.lax.broadcasted_iota(jnp.int32, (BN, D, P, L), 2)
    p_first = pp == 0
    p_last = pp == (P - 1)
    d_first = dd == 0
    d_last = dd == (D - 1)

    up = pltpu.roll(b0, shift=1, axis=2)
    down = pltpu.roll(b1, shift=P - 1, axis=2)
    s2 = a + jnp.where(p_first, zero, up) + jnp.where(p_last, zero, down)

    plus_d = jnp.where(d_last, zero, pltpu.roll(s2, shift=D - 1, axis=1))
    minus_d = jnp.where(d_first, zero, pltpu.roll(s2, shift=1, axis=1))
    box = s2 + plus_d + minus_d

    t = (27.0 * g - box > 0.1).astype(jnp.float32)

    xt = x * t
    bce_part = jnp.maximum(x, zero) + jnp.log(1.0 + jnp.exp(-jnp.abs(x)))

    lane = jax.lax.broadcasted_iota(jnp.int32, (1, 128), 1)
    for i in range(BN):
        inter = jnp.sum(xt[i])
        sum_x = jnp.sum(x[i])
        sum_t = jnp.sum(t[i])
        bp = jnp.sum(bce_part[i])
        stats_ref[i] = (jnp.where(lane == 0, inter, zero)
                        + jnp.where(lane == 1, sum_x, zero)
                        + jnp.where(lane == 2, sum_t, zero)
                        + jnp.where(lane == 3, bp, zero))


def kernel(boundary_logits, gtmasks, weight1, weight2):
    N, C, D, H, W = boundary_logits.shape
    assert C == 1 and H % 2 == 0
    P = H // 2
    L = 2 * W
    BN = _BN if N % _BN == 0 else 1

    x = boundary_logits.reshape(N, D, P, L).astype(jnp.float32)
    g = gtmasks.reshape(N, D, P, L).astype(jnp.float32)

    lv = jnp.arange(L)
    wv = lv % W
    hv = lv // W
    near = jnp.abs(wv[:, None] - wv[None, :]) <= 1
    diff_half = hv[:, None] != hv[None, :]
    t2 = near.astype(jnp.bfloat16)
    tc0 = (near & diff_half & (hv[None, :] == 0)).astype(jnp.bfloat16)
    tc1 = (near & diff_half & (hv[None, :] == 1)).astype(jnp.bfloat16)

    body = functools.partial(_stats_kernel, BN=BN, D=D, P=P, L=L)
    stats = pl.pallas_call(
        body,
        out_shape=jax.ShapeDtypeStruct((N, 1, 128), jnp.float32),
        grid_spec=pltpu.PrefetchScalarGridSpec(
            num_scalar_prefetch=0,
            grid=(N // BN,),
            in_specs=[
                pl.BlockSpec((BN, D, P, L), lambda n: (n, 0, 0, 0)),
                pl.BlockSpec((BN, D, P, L), lambda n: (n, 0, 0, 0)),
                pl.BlockSpec((L, L), lambda n: (0, 0)),
                pl.BlockSpec((L, L), lambda n: (0, 0)),
                pl.BlockSpec((L, L), lambda n: (0, 0)),
            ],
            out_specs=pl.BlockSpec((BN, 1, 128), lambda n: (n, 0, 0)),
        ),
        compiler_params=pltpu.CompilerParams(
            dimension_semantics=("parallel",),
            vmem_limit_bytes=56 * 1024 * 1024,
        ),
    )(x, g, t2, tc0, tc1)

    inter = stats[:, 0, 0]
    sum_x = stats[:, 0, 1]
    sum_t = stats[:, 0, 2]
    bce_sum = stats[:, 0, 3] - inter

    eps = 1.0
    dice_coeff = jnp.mean(2.0 * inter / (sum_x + sum_t + eps))
    dice_loss = 1.0 - dice_coeff
    bce_loss = jnp.sum(bce_sum) / float(N * D * H * W)

    w1 = jnp.asarray(weight1, jnp.float32)
    w2 = jnp.asarray(weight2, jnp.float32)
    return (w1 ** -2) * bce_loss + (w2 ** -2) * dice_loss + jnp.log(1.0 + w1 * w2)


# speedup vs baseline: 2.5411x; 2.5411x over previous
import functools

import jax
import jax.numpy as jnp
from jax.experimental import pallas as pl
from jax.experimental.pallas import tpu as pltpu

_BN = 8


def _stats_kernel(x_ref, g_ref, t_ref, stats_ref, *, BN, D, H, W):
    P = H // 2
    L = 2 * W
    zero = jnp.float32(0.0)

    xn = x_ref[...].reshape(BN, D, H, W)
    gn = g_ref[...].reshape(BN, D, H, W)
    x = jnp.concatenate([xn[:, :, 0:P, :], xn[:, :, P:H, :]], axis=-1)
    g = jnp.concatenate([gn[:, :, 0:P, :], gn[:, :, P:H, :]], axis=-1)

    g2 = g.reshape(BN * D * P, L).astype(jnp.bfloat16)
    s = jax.lax.dot_general(g2, t_ref[...], (((1,), (0,)), ((), ())),
                            preferred_element_type=jnp.float32)
    s = s.reshape(BN, D, P, L)

    dd = jax.lax.broadcasted_iota(jnp.int32, (BN, D, P, L), 1)
    pp = jax.lax.broadcasted_iota(jnp.int32, (BN, D, P, L), 2)
    p_first = pp == 0
    p_last = pp == (P - 1)
    d_first = dd == 0
    d_last = dd == (D - 1)
    lane = jax.lax.broadcasted_iota(jnp.int32, (1, 1, 1, L), 3)
    hm1 = (lane >= W).astype(jnp.float32)
    hm0 = (lane < W).astype(jnp.float32)

    u = pltpu.roll(s, shift=1, axis=2)
    d_ = pltpu.roll(s, shift=P - 1, axis=2)
    u64 = pltpu.roll(u, shift=W, axis=3)
    d64 = pltpu.roll(d_, shift=W, axis=3)
    up = jnp.where(p_first, u64 * hm1, u)
    down = jnp.where(p_last, d64 * hm0, d_)
    s2 = s + up + down

    plus_d = jnp.where(d_last, zero, pltpu.roll(s2, shift=D - 1, axis=1))
    minus_d = jnp.where(d_first, zero, pltpu.roll(s2, shift=1, axis=1))
    box = s2 + plus_d + minus_d

    t = (27.0 * g - box > 0.1).astype(jnp.float32)

    xt = x * t
    bce_part = jnp.maximum(x, zero) + jnp.log(1.0 + jnp.exp(-jnp.abs(x)))

    olane = jax.lax.broadcasted_iota(jnp.int32, (1, 128), 1)
    for i in range(BN):
        inter = jnp.sum(xt[i])
        sum_x = jnp.sum(x[i])
        sum_t = jnp.sum(t[i])
        bp = jnp.sum(bce_part[i])
        stats_ref[i] = (jnp.where(olane == 0, inter, zero)
                        + jnp.where(olane == 1, sum_x, zero)
                        + jnp.where(olane == 2, sum_t, zero)
                        + jnp.where(olane == 3, bp, zero))


def kernel(boundary_logits, gtmasks, weight1, weight2):
    N, C, D, H, W = boundary_logits.shape
    assert C == 1 and H % 2 == 0
    L = 2 * W
    BN = _BN if N % _BN == 0 else 1

    lv = jnp.arange(L)
    wv = lv % W
    hv = lv // W
    band = (jnp.abs(wv[:, None] - wv[None, :]) <= 1) & (hv[:, None] == hv[None, :])
    band = band.astype(jnp.bfloat16)

    body = functools.partial(_stats_kernel, BN=BN, D=D, H=H, W=W)
    stats = pl.pallas_call(
        body,
        out_shape=jax.ShapeDtypeStruct((N, 1, 128), jnp.float32),
        grid_spec=pltpu.PrefetchScalarGridSpec(
            num_scalar_prefetch=0,
            grid=(N // BN,),
            in_specs=[
                pl.BlockSpec((BN, 1, D, H, W), lambda n: (n, 0, 0, 0, 0)),
                pl.BlockSpec((BN, 1, D, H, W), lambda n: (n, 0, 0, 0, 0)),
                pl.BlockSpec((L, L), lambda n: (0, 0)),
            ],
            out_specs=pl.BlockSpec((BN, 1, 128), lambda n: (n, 0, 0)),
        ),
        compiler_params=pltpu.CompilerParams(
            dimension_semantics=("parallel",),
            vmem_limit_bytes=56 * 1024 * 1024,
        ),
    )(boundary_logits, gtmasks, band)

    inter = stats[:, 0, 0]
    sum_x = stats[:, 0, 1]
    sum_t = stats[:, 0, 2]
    bce_sum = stats[:, 0, 3] - inter

    eps = 1.0
    dice_coeff = jnp.mean(2.0 * inter / (sum_x + sum_t + eps))
    dice_loss = 1.0 - dice_coeff
    bce_loss = jnp.sum(bce_sum) / float(N * D * H * W)

    w1 = jnp.asarray(weight1, jnp.float32)
    w2 = jnp.asarray(weight2, jnp.float32)
    return (w1 ** -2) * bce_loss + (w2 ** -2) * dice_loss + jnp.log(1.0 + w1 * w2)


# speedup vs baseline: 2.6017x; 1.0239x over previous
import functools

import numpy as np
import jax
import jax.numpy as jnp
from jax.experimental import pallas as pl
from jax.experimental.pallas import tpu as pltpu

_BN = 8


def _stats_kernel(x_ref, g_ref, t_ref, acc_ref, *, BN, D, H, W):
    P = H // 2
    L = 2 * W
    zero = jnp.float32(0.0)

    xn = x_ref[...].reshape(BN, D, H, W)
    gn = g_ref[...].reshape(BN, D, H, W)
    x = jnp.concatenate([xn[:, :, 0:P, :], xn[:, :, P:H, :]], axis=-1)
    g = jnp.concatenate([gn[:, :, 0:P, :], gn[:, :, P:H, :]], axis=-1)

    g2 = g.reshape(BN * D * P, L).astype(jnp.bfloat16)
    s = jax.lax.dot_general(g2, t_ref[...], (((1,), (0,)), ((), ())),
                            preferred_element_type=jnp.float32)
    s = s.reshape(BN, D, P, L)

    dd = jax.lax.broadcasted_iota(jnp.int32, (BN, D, P, L), 1)
    pp = jax.lax.broadcasted_iota(jnp.int32, (BN, D, P, L), 2)
    p_first = pp == 0
    p_last = pp == (P - 1)
    d_first = dd == 0
    d_last = dd == (D - 1)
    lane = jax.lax.broadcasted_iota(jnp.int32, (1, 1, 1, L), 3)
    hm1 = (lane >= W).astype(jnp.float32)
    hm0 = (lane < W).astype(jnp.float32)

    u = pltpu.roll(s, shift=1, axis=2)
    d_ = pltpu.roll(s, shift=P - 1, axis=2)
    u64 = pltpu.roll(u, shift=W, axis=3)
    d64 = pltpu.roll(d_, shift=W, axis=3)
    up = jnp.where(p_first, u64 * hm1, u)
    down = jnp.where(p_last, d64 * hm0, d_)
    s2 = s + up + down

    plus_d = jnp.where(d_last, zero, pltpu.roll(s2, shift=D - 1, axis=1))
    minus_d = jnp.where(d_first, zero, pltpu.roll(s2, shift=1, axis=1))
    box = s2 + plus_d + minus_d

    t = (27.0 * g - box > 0.1).astype(jnp.float32)

    xt = x * t
    bce_part = jnp.maximum(x, zero) + jnp.log(1.0 + jnp.exp(-jnp.abs(x)))

    @pl.when(pl.program_id(0) == 0)
    def _init():
        acc_ref[...] = jnp.zeros_like(acc_ref)

    olane = jax.lax.broadcasted_iota(jnp.int32, (1, 1, 128), 2)
    row = jnp.zeros((1, 1, 128), jnp.float32)
    for i in range(BN):
        inter = jnp.sum(xt[i])
        sum_x = jnp.sum(x[i])
        sum_t = jnp.sum(t[i])
        bp = jnp.sum(bce_part[i])
        dice_i = 2.0 * inter / (sum_x + sum_t + 1.0)
        bce_i = bp - inter
        row = row + (jnp.where(olane == 0, dice_i, zero)
                     + jnp.where(olane == 1, bce_i, zero))
    acc_ref[...] += row


def kernel(boundary_logits, gtmasks, weight1, weight2):
    N, C, D, H, W = boundary_logits.shape
    assert C == 1 and H % 2 == 0
    L = 2 * W
    BN = _BN if N % _BN == 0 else 1

    lv = np.arange(L)
    wv = lv % W
    hv = lv // W
    band_np = (np.abs(wv[:, None] - wv[None, :]) <= 1) & (hv[:, None] == hv[None, :])
    band = jnp.asarray(band_np, dtype=jnp.bfloat16)

    body = functools.partial(_stats_kernel, BN=BN, D=D, H=H, W=W)
    acc = pl.pallas_call(
        body,
        out_shape=jax.ShapeDtypeStruct((1, 1, 128), jnp.float32),
        grid_spec=pltpu.PrefetchScalarGridSpec(
            num_scalar_prefetch=0,
            grid=(N // BN,),
            in_specs=[
                pl.BlockSpec((BN, 1, D, H, W), lambda n: (n, 0, 0, 0, 0)),
                pl.BlockSpec((BN, 1, D, H, W), lambda n: (n, 0, 0, 0, 0)),
                pl.BlockSpec((L, L), lambda n: (0, 0)),
            ],
            out_specs=pl.BlockSpec((1, 1, 128), lambda n: (0, 0, 0)),
        ),
        compiler_params=pltpu.CompilerParams(
            dimension_semantics=("arbitrary",),
            vmem_limit_bytes=56 * 1024 * 1024,
        ),
    )(boundary_logits, gtmasks, band)

    dice_loss = 1.0 - acc[0, 0, 0] / float(N)
    bce_loss = acc[0, 0, 1] / float(N * D * H * W)

    w1 = jnp.asarray(weight1, jnp.float32)
    w2 = jnp.asarray(weight2, jnp.float32)
    return (w1 ** -2) * bce_loss + (w2 ** -2) * dice_loss + jnp.log(1.0 + w1 * w2)


# speedup vs baseline: 2.8426x; 1.0926x over previous
import functools

import numpy as np
import jax
import jax.numpy as jnp
from jax.experimental import pallas as pl
from jax.experimental.pallas import tpu as pltpu

_BN = 8


def _stats_kernel(x_ref, g_ref, t_ref, w1_ref, w2_ref, out_ref, acc_ref, *,
                  BN, D, H, W, NSTEPS):
    P = H // 2
    L = 2 * W
    N = BN * NSTEPS
    zero = jnp.float32(0.0)

    xn = x_ref[...].reshape(BN, D, H, W)
    gn = g_ref[...].reshape(BN, D, H, W)
    x = jnp.concatenate([xn[:, :, 0:P, :], xn[:, :, P:H, :]], axis=-1)
    g = jnp.concatenate([gn[:, :, 0:P, :], gn[:, :, P:H, :]], axis=-1)

    g2 = g.reshape(BN * D * P, L).astype(jnp.bfloat16)
    s = jax.lax.dot_general(g2, t_ref[...], (((1,), (0,)), ((), ())),
                            preferred_element_type=jnp.float32)
    s = s.reshape(BN, D, P, L)

    dd = jax.lax.broadcasted_iota(jnp.int32, (BN, D, P, L), 1)
    pp = jax.lax.broadcasted_iota(jnp.int32, (BN, D, P, L), 2)
    p_first = pp == 0
    p_last = pp == (P - 1)
    d_first = dd == 0
    d_last = dd == (D - 1)
    lane = jax.lax.broadcasted_iota(jnp.int32, (1, 1, 1, L), 3)
    hm1 = (lane >= W).astype(jnp.float32)
    hm0 = (lane < W).astype(jnp.float32)

    u = pltpu.roll(s, shift=1, axis=2)
    d_ = pltpu.roll(s, shift=P - 1, axis=2)
    u64 = pltpu.roll(u, shift=W, axis=3)
    d64 = pltpu.roll(d_, shift=W, axis=3)
    up = jnp.where(p_first, u64 * hm1, u)
    down = jnp.where(p_last, d64 * hm0, d_)
    s2 = s + up + down

    plus_d = jnp.where(d_last, zero, pltpu.roll(s2, shift=D - 1, axis=1))
    minus_d = jnp.where(d_first, zero, pltpu.roll(s2, shift=1, axis=1))
    box = s2 + plus_d + minus_d

    t = (27.0 * g - box > 0.1).astype(jnp.float32)

    xt = x * t
    bce_part = jnp.maximum(x, zero) + jnp.log(1.0 + jnp.exp(-jnp.abs(x)))

    @pl.when(pl.program_id(0) == 0)
    def _init():
        acc_ref[...] = jnp.zeros_like(acc_ref)

    olane = jax.lax.broadcasted_iota(jnp.int32, (1, 1, 128), 2)
    row = jnp.zeros((1, 1, 128), jnp.float32)
    for i in range(BN):
        inter = jnp.sum(xt[i])
        sum_x = jnp.sum(x[i])
        sum_t = jnp.sum(t[i])
        bp = jnp.sum(bce_part[i])
        dice_i = 2.0 * inter / (sum_x + sum_t + 1.0)
        bce_i = bp - inter
        row = row + (jnp.where(olane == 0, dice_i, zero)
                     + jnp.where(olane == 1, bce_i, zero))
    acc_ref[...] += row

    @pl.when(pl.program_id(0) == NSTEPS - 1)
    def _finish():
        a = acc_ref[...]
        b = pltpu.roll(a, shift=127, axis=2)
        w1 = w1_ref[0] + zero
        w2 = w2_ref[0] + zero
        dice_loss = 1.0 - a * (1.0 / N)
        bce_loss = b * (1.0 / (N * D * H * W))
        loss_row = (bce_loss / (w1 * w1) + dice_loss / (w2 * w2)
                    + jnp.log(1.0 + w1 * w2))
        l0 = (olane == 0).astype(jnp.float32)
        out_ref[0] = jnp.sum(loss_row * l0)


def kernel(boundary_logits, gtmasks, weight1, weight2):
    N, C, D, H, W = boundary_logits.shape
    assert C == 1 and H % 2 == 0
    L = 2 * W
    BN = _BN if N % _BN == 0 else 1

    lv = np.arange(L)
    wv = lv % W
    hv = lv // W
    band_np = (np.abs(wv[:, None] - wv[None, :]) <= 1) & (hv[:, None] == hv[None, :])
    band = jnp.asarray(band_np, dtype=jnp.bfloat16)

    w1 = jnp.asarray(weight1, jnp.float32).reshape(1)
    w2 = jnp.asarray(weight2, jnp.float32).reshape(1)

    body = functools.partial(_stats_kernel, BN=BN, D=D, H=H, W=W,
                             NSTEPS=N // BN)
    out = pl.pallas_call(
        body,
        out_shape=jax.ShapeDtypeStruct((1,), jnp.float32),
        grid_spec=pltpu.PrefetchScalarGridSpec(
            num_scalar_prefetch=0,
            grid=(N // BN,),
            in_specs=[
                pl.BlockSpec((BN, 1, D, H, W), lambda n: (n, 0, 0, 0, 0)),
                pl.BlockSpec((BN, 1, D, H, W), lambda n: (n, 0, 0, 0, 0)),
                pl.BlockSpec((L, L), lambda n: (0, 0)),
                pl.BlockSpec(memory_space=pltpu.SMEM),
                pl.BlockSpec(memory_space=pltpu.SMEM),
            ],
            out_specs=pl.BlockSpec(memory_space=pltpu.SMEM),
            scratch_shapes=[pltpu.VMEM((1, 1, 128), jnp.float32)],
        ),
        compiler_params=pltpu.CompilerParams(
            dimension_semantics=("arbitrary",),
            vmem_limit_bytes=56 * 1024 * 1024,
        ),
    )(boundary_logits, gtmasks, band, w1, w2)

    return out
